# Initial kernel scaffold; baseline (speedup 1.0000x reference)
import jax
import jax.numpy as jnp
from jax.experimental import pallas as pl


def kernel(x, edge_index, edge_attr, batch, epoch, W1e, W1m, W1s, A1, b1, W2e, W2m, W2s, A2, b2, Wc, bc):
    raise NotImplementedError("write your pallas kernel here")


# speedup vs baseline: 2.5173x; 2.5173x over previous
import functools
import jax
import jax.numpy as jnp
from jax import lax
from jax.experimental import pallas as pl
from jax.experimental.pallas import tpu as pltpu
from jax.experimental.pallas import tpu_sc as plsc

N = 10000
E = 320000
D = 128
G = 128
CHUNK = 80
NROW = E // CHUNK
NC, NS = 2, 16
NZ = 1000


def _mm_body(x_ref, wm_ref, ws_ref, xm_ref, xs_ref):
    xb = x_ref[...]
    xm_ref[...] = jnp.dot(xb, wm_ref[0], preferred_element_type=jnp.float32)
    xs_ref[...] = jnp.dot(xb, ws_ref[0], preferred_element_type=jnp.float32)


def _node_tables(x, wm_stack, ws_stack):
    return pl.pallas_call(
        _mm_body,
        grid=(20,),
        in_specs=[
            pl.BlockSpec((1000, D), lambda j: (j % 10, 0)),
            pl.BlockSpec((1, D, D), lambda j: (j // 10, 0, 0)),
            pl.BlockSpec((1, D, D), lambda j: (j // 10, 0, 0)),
        ],
        out_specs=[
            pl.BlockSpec((1000, D), lambda j: (j, 0)),
            pl.BlockSpec((1000, D), lambda j: (j, 0)),
        ],
        out_shape=[
            jax.ShapeDtypeStruct((2 * N, D), jnp.float32),
            jax.ShapeDtypeStruct((2 * N, D), jnp.float32),
        ],
    )(x, wm_stack, ws_stack)


def _sc_pass12(xm_stack, src2d, dst2d, ea, ww_stack, zeros_nd):
    mesh = plsc.VectorSubcoreMesh(core_axis_name="c", subcore_axis_name="s")
    rows_per = NROW // NS

    @functools.partial(
        pl.kernel,
        out_type=jax.ShapeDtypeStruct((2 * N, D), jnp.float32),
        mesh=mesh,
        compiler_params=pltpu.CompilerParams(needs_layout_passes=False),
        scratch_types=[
            pltpu.VMEM_SHARED((N, D), jnp.float32),
            pltpu.VMEM((CHUNK,), jnp.int32),
            pltpu.VMEM((CHUNK,), jnp.int32),
            pltpu.VMEM((4 * CHUNK + 16,), jnp.float32),
            pltpu.VMEM((CHUNK, D), jnp.float32),
            pltpu.VMEM((4 * D,), jnp.float32),
            pltpu.SemaphoreType.DMA,
        ],
    )
    def k(xm_hbm, src_hbm, dst_hbm, ea_hbm, ww_hbm, z_hbm, agg_hbm,
          agg_sp, src_v, dst_v, ea_v, rows_v, w_v, sem):
        cid = lax.axis_index("c")
        sid = lax.axis_index("s")
        zbase = pl.multiple_of(sid * NZ, 8)

        @pl.when(sid < N // NZ)
        def _zero():
            pltpu.sync_copy(z_hbm.at[pl.ds(zbase, NZ)],
                            agg_sp.at[pl.ds(zbase, NZ)])

        wbase = pl.multiple_of(cid * 4 * D, 8)
        pltpu.sync_copy(ww_hbm.at[pl.ds(wbase, 4 * D)], w_v)
        plsc.subcore_barrier()

        wv = [[w_v[pl.ds(k_ * D + 16 * v, 16)] for k_ in range(4)]
              for v in range(8)]
        off = cid * N

        def chunk_body(r, _):
            row = sid * rows_per + r
            ebase = pl.multiple_of(row * CHUNK, 8)
            pltpu.sync_copy(src_hbm.at[pl.ds(ebase, CHUNK)], src_v)
            pltpu.sync_copy(dst_hbm.at[pl.ds(ebase, CHUNK)], dst_v)
            pltpu.sync_copy(ea_hbm.at[pl.ds(row * 4 * CHUNK, 4 * CHUNK)],
                            ea_v.at[pl.ds(0, 4 * CHUNK)])
            for g in range(CHUNK // 16):
                sl = pl.ds(g * 16, 16)
                src_v[sl] = src_v[sl] + off
            pltpu.async_copy(xm_hbm.at[src_v], rows_v, sem).wait()

            def edge_body(e, _):
                av = ea_v[pl.ds(4 * e, 16)]
                a0 = av[0]
                a1 = av[1]
                a2 = av[2]
                a3 = av[3]
                for v in range(8):
                    sl = pl.ds(16 * v, 16)
                    t = rows_v[e, sl]
                    t = t + a0 * wv[v][0] + a1 * wv[v][1]
                    t = t + a2 * wv[v][2] + a3 * wv[v][3]
                    rows_v[e, sl] = jnp.maximum(t, 0.0)
                return ()

            lax.fori_loop(0, CHUNK, edge_body, ())
            pltpu.sync_copy(rows_v, agg_sp.at[dst_v], add=True)
            return ()

        lax.fori_loop(0, rows_per, chunk_body, ())
        plsc.subcore_barrier()

        @pl.when(sid < N // NZ)
        def _flush():
            obase = pl.multiple_of(cid * N + sid * NZ, 8)
            pltpu.sync_copy(agg_sp.at[pl.ds(zbase, NZ)],
                            agg_hbm.at[pl.ds(obase, NZ)])

    return k(xm_stack, src2d, dst2d, ea, ww_stack, zeros_nd)


def _h_body(xs1_ref, a1_ref, xs2_ref, a2_ref, A1_ref, A2_ref,
            b1_ref, b2_ref, sd_ref):
    h1 = jnp.maximum(xs1_ref[...] + a1_ref[...], 0.0)
    h2 = jnp.maximum(xs2_ref[...] + a2_ref[...], 0.0)
    A1 = A1_ref[...]
    A2 = A2_ref[...]
    s1 = jnp.dot(h1, A1[:D], preferred_element_type=jnp.float32) + b1_ref[0]
    d1 = jnp.dot(h1, A1[D:], preferred_element_type=jnp.float32)
    s2 = jnp.dot(h2, A2[:D], preferred_element_type=jnp.float32) + b2_ref[0]
    d2 = jnp.dot(h2, A2[D:], preferred_element_type=jnp.float32)
    pad = jnp.zeros((1000, D - 4), jnp.float32)
    sd_ref[...] = jnp.concatenate([s1, d1, s2, d2, pad], axis=1)


def _edge_logit_tables(xs_stack, agg12, A1, b1, A2, b2):
    return pl.pallas_call(
        _h_body,
        grid=(10,),
        in_specs=[
            pl.BlockSpec((1000, D), lambda i: (i, 0)),
            pl.BlockSpec((1000, D), lambda i: (i, 0)),
            pl.BlockSpec((1000, D), lambda i: (10 + i, 0)),
            pl.BlockSpec((1000, D), lambda i: (10 + i, 0)),
            pl.BlockSpec((2 * D, 1), lambda i: (0, 0)),
            pl.BlockSpec((2 * D, 1), lambda i: (0, 0)),
            pl.BlockSpec(memory_space=pltpu.SMEM),
            pl.BlockSpec(memory_space=pltpu.SMEM),
        ],
        out_specs=pl.BlockSpec((1000, D), lambda i: (i, 0)),
        out_shape=jax.ShapeDtypeStruct((N, D), jnp.float32),
    )(xs_stack, agg12, xs_stack, agg12, A1, A2, b1, b2)


def _sc_pass3(xm_stack, src2d, dst2d, ea, ww_stack, zeros_nd, s1, d1, s2, d2):
    mesh = plsc.VectorSubcoreMesh(core_axis_name="c", subcore_axis_name="s")
    rows_per = NROW // (NC * NS)

    @functools.partial(
        pl.kernel,
        out_type=[
            jax.ShapeDtypeStruct((E,), jnp.float32),
            jax.ShapeDtypeStruct((E,), jnp.float32),
        ],
        mesh=mesh,
        compiler_params=pltpu.CompilerParams(needs_layout_passes=False),
        scratch_types=[
            pltpu.VMEM((CHUNK,), jnp.int32),
            pltpu.VMEM((CHUNK,), jnp.int32),
            pltpu.VMEM((CHUNK,), jnp.float32),
            pltpu.VMEM((CHUNK,), jnp.float32),
            pltpu.VMEM((N,), jnp.float32),
            pltpu.VMEM((N,), jnp.float32),
            pltpu.VMEM((N,), jnp.float32),
            pltpu.VMEM((N,), jnp.float32),
        ],
    )
    def katt(src_hbm, dst_hbm, s1_hbm, d1_hbm, s2_hbm, d2_hbm,
             att_hbm, em_hbm,
             src_v, dst_v, att_v, em_v, s1_v, d1_v, s2_v, d2_v):
        cid = lax.axis_index("c")
        sid = lax.axis_index("s")
        wid = cid * NS + sid
        pltpu.sync_copy(s1_hbm, s1_v)
        pltpu.sync_copy(d1_hbm, d1_v)
        pltpu.sync_copy(s2_hbm, s2_v)
        pltpu.sync_copy(d2_hbm, d2_v)

        def sigmoid(x):
            return 1.0 / (1.0 + jnp.exp(-x))

        def chunk_body(r, _):
            row = wid * rows_per + r
            ebase = pl.multiple_of(row * CHUNK, 8)
            pltpu.sync_copy(src_hbm.at[pl.ds(ebase, CHUNK)], src_v)
            pltpu.sync_copy(dst_hbm.at[pl.ds(ebase, CHUNK)], dst_v)
            for g in range(CHUNK // 16):
                sl = pl.ds(g * 16, 16)
                sv = src_v[sl]
                dv = dst_v[sl]
                a1 = sigmoid(plsc.load_gather(s1_v, [sv])
                             + plsc.load_gather(d1_v, [dv]))
                a2 = sigmoid(plsc.load_gather(s2_v, [sv])
                             + plsc.load_gather(d2_v, [dv]))
                att_v[sl] = a2
                em_v[sl] = a1 * a2
            pltpu.sync_copy(att_v, att_hbm.at[pl.ds(ebase, CHUNK)])
            pltpu.sync_copy(em_v, em_hbm.at[pl.ds(ebase, CHUNK)])
            return ()

        lax.fori_loop(0, rows_per, chunk_body, ())

    att_flat, em_flat = katt(src2d, dst2d, s1, d1, s2, d2)

    @functools.partial(
        pl.kernel,
        out_type=jax.ShapeDtypeStruct((2 * N, D), jnp.float32),
        mesh=mesh,
        compiler_params=pltpu.CompilerParams(needs_layout_passes=False),
        scratch_types=[
            pltpu.VMEM_SHARED((N, D), jnp.float32),
            pltpu.VMEM((CHUNK,), jnp.int32),
            pltpu.VMEM((CHUNK,), jnp.int32),
            pltpu.VMEM((4 * CHUNK + 16,), jnp.float32),
            pltpu.VMEM((CHUNK, D), jnp.float32),
            pltpu.VMEM((CHUNK + 16,), jnp.float32),
            pltpu.VMEM((4 * D,), jnp.float32),
            pltpu.SemaphoreType.DMA,
        ],
    )
    def k(xm_hbm, src_hbm, dst_hbm, ea_hbm, ww_hbm, z_hbm, em_hbm, agg_hbm,
          agg_sp, src_v, dst_v, ea_v, rows_v, em_v, w_v, sem):
        cid = lax.axis_index("c")
        sid = lax.axis_index("s")
        wid = cid * NS + sid
        zbase = pl.multiple_of(sid * NZ, 8)

        @pl.when(sid < N // NZ)
        def _zero():
            pltpu.sync_copy(z_hbm.at[pl.ds(zbase, NZ)],
                            agg_sp.at[pl.ds(zbase, NZ)])

        pltpu.sync_copy(ww_hbm.at[pl.ds(4 * D, 4 * D)], w_v)
        plsc.subcore_barrier()

        wv = [[w_v[pl.ds(k_ * D + 16 * v, 16)] for k_ in range(4)]
              for v in range(8)]

        def chunk_body(r, _):
            row = wid * rows_per + r
            ebase = pl.multiple_of(row * CHUNK, 8)
            pltpu.sync_copy(src_hbm.at[pl.ds(ebase, CHUNK)], src_v)
            pltpu.sync_copy(dst_hbm.at[pl.ds(ebase, CHUNK)], dst_v)
            pltpu.sync_copy(ea_hbm.at[pl.ds(row * 4 * CHUNK, 4 * CHUNK)],
                            ea_v.at[pl.ds(0, 4 * CHUNK)])
            pltpu.sync_copy(em_hbm.at[pl.ds(ebase, CHUNK)],
                            em_v.at[pl.ds(0, CHUNK)])
            for g in range(CHUNK // 16):
                sl = pl.ds(g * 16, 16)
                src_v[sl] = src_v[sl] + N
            pltpu.async_copy(xm_hbm.at[src_v], rows_v, sem).wait()

            def edge_body(e, _):
                av = ea_v[pl.ds(4 * e, 16)]
                a0 = av[0]
                a1 = av[1]
                a2 = av[2]
                a3 = av[3]
                m = em_v[pl.ds(e, 16)][0]
                for v in range(8):
                    sl = pl.ds(16 * v, 16)
                    t = rows_v[e, sl]
                    t = t + a0 * wv[v][0] + a1 * wv[v][1]
                    t = t + a2 * wv[v][2] + a3 * wv[v][3]
                    rows_v[e, sl] = jnp.maximum(t, 0.0) * m
                return ()

            lax.fori_loop(0, CHUNK, edge_body, ())
            pltpu.sync_copy(rows_v, agg_sp.at[dst_v], add=True)
            return ()

        lax.fori_loop(0, rows_per, chunk_body, ())
        plsc.subcore_barrier()

        @pl.when(sid < N // NZ)
        def _flush():
            obase = pl.multiple_of(cid * N + sid * NZ, 8)
            pltpu.sync_copy(agg_sp.at[pl.ds(zbase, NZ)],
                            agg_hbm.at[pl.ds(obase, NZ)])

    agg3 = k(xm_stack, src2d, dst2d, ea, ww_stack, zeros_nd, em_flat)
    return att_flat, agg3


def _pool_body(xs2_ref, a_ref, b_ref, batch_ref, wc_ref, bc_ref, out_ref,
               acc_ref, cnt_ref):
    i = pl.program_id(0)
    h = jnp.maximum(xs2_ref[...] + a_ref[...] + b_ref[...], 0.0)
    bb = batch_ref[...]
    gids = lax.broadcasted_iota(jnp.int32, (1000, G), 1)
    oh = (bb == gids).astype(jnp.float32)
    psum = lax.dot_general(oh, h, (((0,), (0,)), ((), ())),
                           preferred_element_type=jnp.float32)
    csum = lax.dot_general(oh, jnp.ones((1000, 2), jnp.float32),
                           (((0,), (0,)), ((), ())),
                           preferred_element_type=jnp.float32)

    @pl.when(i == 0)
    def _init():
        acc_ref[...] = jnp.zeros_like(acc_ref)
        cnt_ref[...] = jnp.zeros_like(cnt_ref)

    acc_ref[...] += psum
    cnt_ref[...] += csum

    @pl.when(i == pl.num_programs(0) - 1)
    def _fin():
        raw = jnp.dot(acc_ref[...], wc_ref[...],
                      preferred_element_type=jnp.float32)
        out_ref[...] = raw / jnp.maximum(cnt_ref[...], 1.0) + bc_ref[...]


def _pool(xs_stack, agg3, batch2d, wc, bc2d):
    return pl.pallas_call(
        _pool_body,
        grid=(10,),
        in_specs=[
            pl.BlockSpec((1000, D), lambda i: (10 + i, 0)),
            pl.BlockSpec((1000, D), lambda i: (i, 0)),
            pl.BlockSpec((1000, D), lambda i: (10 + i, 0)),
            pl.BlockSpec((1000, 1), lambda i: (i, 0)),
            pl.BlockSpec((D, 2), lambda i: (0, 0)),
            pl.BlockSpec((1, 2), lambda i: (0, 0)),
        ],
        out_specs=pl.BlockSpec((G, 2), lambda i: (0, 0)),
        out_shape=jax.ShapeDtypeStruct((G, 2), jnp.float32),
        scratch_shapes=[
            pltpu.VMEM((G, G), jnp.float32),
            pltpu.VMEM((G, 2), jnp.float32),
        ],
    )(xs_stack, agg3, agg3, batch2d, wc, bc2d)


def kernel(x, edge_index, edge_attr, batch, epoch,
           W1e, W1m, W1s, A1, b1, W2e, W2m, W2s, A2, b2, Wc, bc):
    src_flat = edge_index[0].astype(jnp.int32)
    dst_flat = edge_index[1].astype(jnp.int32)
    ea_flat = edge_attr.reshape(E * 4)
    wm_stack = jnp.stack([W1m, W2m])
    ws_stack = jnp.stack([W1s, W2s])
    ww_stack = jnp.concatenate([W1e @ W1m, W2e @ W2m], axis=0).reshape(8 * D)
    zeros_nd = jnp.zeros((N, D), jnp.float32)

    xm_stack, xs_stack = _node_tables(x, wm_stack, ws_stack)
    agg12 = _sc_pass12(xm_stack, src_flat, dst_flat, ea_flat, ww_stack,
                       zeros_nd)
    sd = _edge_logit_tables(xs_stack, agg12, A1, b1, A2, b2)
    s1, d1, s2, d2 = (sd[:, 0], sd[:, 1], sd[:, 2], sd[:, 3])
    att_flat, agg3 = _sc_pass3(xm_stack, src_flat, dst_flat, ea_flat,
                               ww_stack, zeros_nd, s1, d1, s2, d2)
    logits = _pool(xs_stack, agg3, batch.astype(jnp.int32).reshape(N, 1),
                   Wc, bc.reshape(1, 2))
    return (att_flat.reshape(E, 1), logits)
